# Initial kernel scaffold; baseline (speedup 1.0000x reference)
import jax
import jax.numpy as jnp
from jax.experimental import pallas as pl


def kernel(x, conv1_w, c1_gamma, c1_beta, lin1_w, lin1_b, bn1_gamma, bn1_beta, lin2_w, lin2_b, bn2_gamma, bn2_beta, lin3_w, lin3_b):
    raise NotImplementedError("write your pallas kernel here")


# speedup vs baseline: 11.7963x; 11.7963x over previous
import jax
import jax.numpy as jnp
from jax.experimental import pallas as pl

_EPS = 1e-5
_BN = 256


def _leaky(x):
    return jnp.where(x >= 0, x, 0.2 * x)


def _bn2d(x, gamma, beta):
    mean = jnp.mean(x, axis=(0, 2, 3), keepdims=True)
    var = jnp.var(x, axis=(0, 2, 3), keepdims=True)
    xh = (x - mean) / jnp.sqrt(var + _EPS)
    return xh * gamma.reshape(1, -1, 1, 1) + beta.reshape(1, -1, 1, 1)


def _bn1d(x, gamma, beta):
    mean = jnp.mean(x, axis=0, keepdims=True)
    var = jnp.var(x, axis=0, keepdims=True)
    xh = (x - mean) / jnp.sqrt(var + _EPS)
    return xh * gamma + beta


def _knn_small(x, k):
    inner_prod = -2.0 * jnp.matmul(jnp.transpose(x, (0, 2, 1)), x)
    xx = jnp.sum(x ** 2, axis=1, keepdims=True)
    distances = -xx - inner_prod - jnp.transpose(xx, (0, 2, 1))
    _, idx = jax.lax.top_k(distances, k)
    return idx


def _knn_tile(a_ref, b_ref, xxr_ref, xxc_ref, i1_ref, i2_ref, i3_ref):
    a = a_ref[0]
    bm = b_ref[0]
    xxr = xxr_ref[0]
    xxc = xxc_ref[0]

    dt = jax.lax.dot_general(
        a.astype(jnp.bfloat16), bm.astype(jnp.bfloat16),
        (((0,), (0,)), ((), ())),
        preferred_element_type=jnp.float32,
    )
    inner = -2.0 * dt
    d = (-xxc) - inner - xxr

    bn, n = d.shape
    iota = jax.lax.broadcasted_iota(jnp.int32, (bn, n), 1)
    big = jnp.int32(n)
    neg = jnp.float32(-jnp.inf)

    def pick(dcur):
        v = jnp.max(dcur, axis=1, keepdims=True)
        return jnp.min(jnp.where(dcur == v, iota, big), axis=1, keepdims=True)

    i1 = pick(d)
    d = jnp.where(iota == i1, neg, d)
    i2 = pick(d)
    d = jnp.where(iota == i2, neg, d)
    i3 = pick(d)

    i1_ref[0] = i1
    i2_ref[0] = i2
    i3_ref[0] = i3


def _knn_big(pts):
    B, _, N = pts.shape
    xx = jnp.sum(pts ** 2, axis=1, keepdims=True)
    xxT = jnp.transpose(xx, (0, 2, 1))
    grid = (B, N // _BN)
    outs = pl.pallas_call(
        _knn_tile,
        grid=grid,
        in_specs=[
            pl.BlockSpec((1, 3, _BN), lambda b, i: (b, 0, i)),
            pl.BlockSpec((1, 3, N), lambda b, i: (b, 0, 0)),
            pl.BlockSpec((1, _BN, 1), lambda b, i: (b, i, 0)),
            pl.BlockSpec((1, 1, N), lambda b, i: (b, 0, 0)),
        ],
        out_specs=[
            pl.BlockSpec((1, _BN, 1), lambda b, i: (b, i, 0)),
            pl.BlockSpec((1, _BN, 1), lambda b, i: (b, i, 0)),
            pl.BlockSpec((1, _BN, 1), lambda b, i: (b, i, 0)),
        ],
        out_shape=[
            jax.ShapeDtypeStruct((B, N, 1), jnp.int32),
            jax.ShapeDtypeStruct((B, N, 1), jnp.int32),
            jax.ShapeDtypeStruct((B, N, 1), jnp.int32),
        ],
    )(pts, pts, xxT, xx)
    return jnp.concatenate(outs, axis=-1)


def _graph_feature_from_idx(x, idx, k):
    B, C, N = x.shape
    idx_base = jnp.arange(B).reshape(-1, 1, 1) * N
    idx = (idx + idx_base).reshape(-1)
    feature = jnp.take(x.reshape(B * N, -1), idx, axis=0)
    feature = feature.reshape(B, N, k, C)
    xr = jnp.broadcast_to(x.reshape(B, N, 1, C), (B, N, k, C))
    feature = jnp.concatenate((feature - xr, xr), axis=3)
    return feature.reshape(B, C * 2, k, N)


def _graph_feature_small(x, k):
    batch_size = x.shape[0]
    num_points = x.shape[1]
    x = x.reshape(batch_size, -1, num_points)
    idx = _knn_small(x, k)
    idx_base = jnp.arange(batch_size).reshape(-1, 1, 1) * num_points
    idx = (idx + idx_base).reshape(-1)
    num_dims = x.shape[1]
    feature = jnp.take(x.reshape(batch_size * num_points, -1), idx, axis=0)
    feature = feature.reshape(batch_size, num_points, k, num_dims)
    xr = jnp.broadcast_to(
        x.reshape(batch_size, num_points, 1, num_dims),
        (batch_size, num_points, k, num_dims))
    feature = jnp.concatenate((feature - xr, xr), axis=3)
    return feature.reshape(batch_size, num_dims * 2, k, num_points)


def kernel(x, conv1_w, c1_gamma, c1_beta, lin1_w, lin1_b, bn1_gamma, bn1_beta,
           lin2_w, lin2_b, bn2_gamma, bn2_beta, lin3_w, lin3_b):
    k = 3
    batch_size = x.shape[0]
    num_points = x.shape[1]

    x0 = x.reshape(batch_size, -1, num_points)
    idx = _knn_big(x0)
    feat = _graph_feature_from_idx(x0, idx, k)

    h = jnp.einsum('oc,bckn->bokn', conv1_w, feat)
    h = _leaky(_bn2d(h, c1_gamma, c1_beta))
    x1 = jnp.max(h, axis=-1)
    x2 = jnp.max(_graph_feature_small(x1, k), axis=-1)
    x3 = jnp.max(_graph_feature_small(x2, k), axis=-1)
    x4 = jnp.max(_graph_feature_small(x3, k), axis=-1)
    xc = jnp.concatenate((x1, x2, x3, x4), axis=1)
    pooled = jnp.max(xc, axis=-1).reshape(batch_size, -1)
    h = pooled @ lin1_w.T + lin1_b
    h = _leaky(_bn1d(h, bn1_gamma, bn1_beta))
    h = h @ lin2_w.T + lin2_b
    h = _leaky(_bn1d(h, bn2_gamma, bn2_beta))
    h = h @ lin3_w.T + lin3_b
    return jax.nn.softmax(h, axis=1)


# speedup vs baseline: 12.2789x; 1.0409x over previous
import jax
import jax.numpy as jnp
from jax.experimental import pallas as pl

_EPS = 1e-5
_BN = 512


def _leaky(x):
    return jnp.where(x >= 0, x, 0.2 * x)


def _bn2d(x, gamma, beta):
    mean = jnp.mean(x, axis=(0, 2, 3), keepdims=True)
    var = jnp.var(x, axis=(0, 2, 3), keepdims=True)
    xh = (x - mean) / jnp.sqrt(var + _EPS)
    return xh * gamma.reshape(1, -1, 1, 1) + beta.reshape(1, -1, 1, 1)


def _bn1d(x, gamma, beta):
    mean = jnp.mean(x, axis=0, keepdims=True)
    var = jnp.var(x, axis=0, keepdims=True)
    xh = (x - mean) / jnp.sqrt(var + _EPS)
    return xh * gamma + beta


def _knn_small(x, k):
    inner_prod = -2.0 * jnp.matmul(jnp.transpose(x, (0, 2, 1)), x)
    xx = jnp.sum(x ** 2, axis=1, keepdims=True)
    distances = -xx - inner_prod - jnp.transpose(xx, (0, 2, 1))
    _, idx = jax.lax.top_k(distances, k)
    return idx


def _knn_tile(a_ref, b_ref, xxr_ref, xxc_ref, i1_ref, i2_ref, i3_ref):
    a = a_ref[0]
    bm = b_ref[0]
    xxr = xxr_ref[0]
    xxc = xxc_ref[0]

    dt = jax.lax.dot_general(
        a.astype(jnp.bfloat16), bm.astype(jnp.bfloat16),
        (((0,), (0,)), ((), ())),
        preferred_element_type=jnp.float32,
    )
    inner = -2.0 * dt
    d = (-xxc) - inner - xxr

    bn, n = d.shape
    iota = jax.lax.broadcasted_iota(jnp.int32, (bn, n), 1)
    big = jnp.int32(n)
    neg = jnp.float32(-jnp.inf)

    def pick(dcur):
        v = jnp.max(dcur, axis=1, keepdims=True)
        return jnp.min(jnp.where(dcur == v, iota, big), axis=1, keepdims=True)

    i1 = pick(d)
    d = jnp.where(iota == i1, neg, d)
    i2 = pick(d)
    d = jnp.where(iota == i2, neg, d)
    i3 = pick(d)

    i1_ref[0] = i1
    i2_ref[0] = i2
    i3_ref[0] = i3


def _knn_big(pts):
    B, _, N = pts.shape
    xx = jnp.sum(pts ** 2, axis=1, keepdims=True)
    xxT = jnp.transpose(xx, (0, 2, 1))
    grid = (B, N // _BN)
    outs = pl.pallas_call(
        _knn_tile,
        grid=grid,
        in_specs=[
            pl.BlockSpec((1, 3, _BN), lambda b, i: (b, 0, i)),
            pl.BlockSpec((1, 3, N), lambda b, i: (b, 0, 0)),
            pl.BlockSpec((1, _BN, 1), lambda b, i: (b, i, 0)),
            pl.BlockSpec((1, 1, N), lambda b, i: (b, 0, 0)),
        ],
        out_specs=[
            pl.BlockSpec((1, _BN, 1), lambda b, i: (b, i, 0)),
            pl.BlockSpec((1, _BN, 1), lambda b, i: (b, i, 0)),
            pl.BlockSpec((1, _BN, 1), lambda b, i: (b, i, 0)),
        ],
        out_shape=[
            jax.ShapeDtypeStruct((B, N, 1), jnp.int32),
            jax.ShapeDtypeStruct((B, N, 1), jnp.int32),
            jax.ShapeDtypeStruct((B, N, 1), jnp.int32),
        ],
    )(pts, pts, xxT, xx)
    return jnp.concatenate(outs, axis=-1)


def _graph_feature_from_idx(x, idx, k):
    B, C, N = x.shape
    idx_base = jnp.arange(B).reshape(-1, 1, 1) * N
    idx = (idx + idx_base).reshape(-1)
    feature = jnp.take(x.reshape(B * N, -1), idx, axis=0)
    feature = feature.reshape(B, N, k, C)
    xr = jnp.broadcast_to(x.reshape(B, N, 1, C), (B, N, k, C))
    feature = jnp.concatenate((feature - xr, xr), axis=3)
    return feature.reshape(B, C * 2, k, N)


def _graph_feature_small(x, k):
    batch_size = x.shape[0]
    num_points = x.shape[1]
    x = x.reshape(batch_size, -1, num_points)
    idx = _knn_small(x, k)
    idx_base = jnp.arange(batch_size).reshape(-1, 1, 1) * num_points
    idx = (idx + idx_base).reshape(-1)
    num_dims = x.shape[1]
    feature = jnp.take(x.reshape(batch_size * num_points, -1), idx, axis=0)
    feature = feature.reshape(batch_size, num_points, k, num_dims)
    xr = jnp.broadcast_to(
        x.reshape(batch_size, num_points, 1, num_dims),
        (batch_size, num_points, k, num_dims))
    feature = jnp.concatenate((feature - xr, xr), axis=3)
    return feature.reshape(batch_size, num_dims * 2, k, num_points)


def kernel(x, conv1_w, c1_gamma, c1_beta, lin1_w, lin1_b, bn1_gamma, bn1_beta,
           lin2_w, lin2_b, bn2_gamma, bn2_beta, lin3_w, lin3_b):
    k = 3
    batch_size = x.shape[0]
    num_points = x.shape[1]

    x0 = x.reshape(batch_size, -1, num_points)
    idx = _knn_big(x0)
    feat = _graph_feature_from_idx(x0, idx, k)

    h = jnp.einsum('oc,bckn->bokn', conv1_w, feat)
    h = _leaky(_bn2d(h, c1_gamma, c1_beta))
    x1 = jnp.max(h, axis=-1)
    x2 = jnp.max(_graph_feature_small(x1, k), axis=-1)
    x3 = jnp.max(_graph_feature_small(x2, k), axis=-1)
    x4 = jnp.max(_graph_feature_small(x3, k), axis=-1)
    xc = jnp.concatenate((x1, x2, x3, x4), axis=1)
    pooled = jnp.max(xc, axis=-1).reshape(batch_size, -1)
    h = pooled @ lin1_w.T + lin1_b
    h = _leaky(_bn1d(h, bn1_gamma, bn1_beta))
    h = h @ lin2_w.T + lin2_b
    h = _leaky(_bn1d(h, bn2_gamma, bn2_beta))
    h = h @ lin3_w.T + lin3_b
    return jax.nn.softmax(h, axis=1)
